# baseline (speedup 1.0000x reference)
import jax
import jax.numpy as jnp
from jax.experimental import pallas as pl
from jax.experimental.pallas import tpu as pltpu

_K = 9
_MAX_DILATION = 3
_KD = 18
_OUT_COLS = 16
_IPS = 4


def _knn_body(x_ref, out_ref, xx0, xx1, h0, h1):
    i = pl.program_id(0)
    n = x_ref.shape[1]
    rows = _IPS * n

    def produce(xx_s, h_s):
        for g in range(_IPS):
            xg = x_ref[g]
            xx_s[g * n:(g + 1) * n, :] = jax.lax.dot_general(
                xg, xg, (((1,), (1,)), ((), ())),
                preferred_element_type=jnp.float32,
            )
            h_s[g * n:(g + 1) * n, :] = 0.5 * jnp.sum(xg * xg, axis=1, keepdims=True)

    def consume(xx_s, h_s):
        h = h_s[...]
        row_mod = jax.lax.broadcasted_iota(jnp.int32, (rows, 1), 0) % n
        eye = row_mod == jax.lax.broadcasted_iota(jnp.int32, (rows, n), 1)
        ht = jnp.concatenate(
            [jnp.broadcast_to(jnp.transpose(h[g * n:(g + 1) * n]), (n, n))
             for g in range(_IPS)], axis=0)
        d2h = jnp.where(eye, 0.0, jnp.maximum(h + ht - xx_s[...], 0.0))
        iota_f = jax.lax.broadcasted_iota(jnp.int32, (rows, n), 1).astype(jnp.float32)
        big = jnp.float32(1e30)
        cols = [row_mod.astype(jnp.float32)]
        m = jnp.float32(0.0)
        for k in range(1, _KD):
            m = jnp.min(jnp.where(d2h > m, d2h, big), axis=1, keepdims=True)
            if k % 2 == 0:
                cols.append(
                    jnp.min(jnp.where(d2h == m, iota_f, big), axis=1, keepdims=True)
                )
        cols.append(jnp.zeros((rows, _OUT_COLS - len(cols)), jnp.float32))
        res = jnp.concatenate(cols, axis=1).astype(jnp.int32)
        out_ref[...] = res.reshape(_IPS, n, _OUT_COLS)

    @pl.when(i % 2 == 0)
    def _():
        produce(xx0, h0)
        consume(xx1, h1)

    @pl.when(i % 2 == 1)
    def _():
        produce(xx1, h1)
        consume(xx0, h0)


def kernel(x, layer_idx):
    B, N, C = x.shape
    nsteps = B // _IPS
    idx_pad = pl.pallas_call(
        _knn_body,
        grid=(nsteps + 1,),
        in_specs=[pl.BlockSpec(
            (_IPS, N, C), lambda i: (jnp.minimum(i, nsteps - 1), 0, 0))],
        out_specs=pl.BlockSpec(
            (_IPS, N, _OUT_COLS), lambda i: (jnp.maximum(i - 1, 0), 0, 0)),
        out_shape=jax.ShapeDtypeStruct((B, N, _OUT_COLS), jnp.int32),
        scratch_shapes=[
            pltpu.VMEM((_IPS * N, N), jnp.float32),
            pltpu.VMEM((_IPS * N, N), jnp.float32),
            pltpu.VMEM((_IPS * N, 1), jnp.float32),
            pltpu.VMEM((_IPS * N, 1), jnp.float32),
        ],
    )(x)
    idx9 = idx_pad[:, :, :_K]
    dil_traced = jnp.minimum(layer_idx // 4 + 1, _MAX_DILATION)
    corr = (dil_traced - 2).astype(jnp.int32)
    offsets = (jnp.arange(B, dtype=jnp.int32) * N)[:, None, None]
    src = (idx9 + offsets + corr).reshape(-1)
    dst_iota = jnp.broadcast_to(
        jnp.arange(N, dtype=jnp.int32)[None, :, None], (B, N, _K)
    )
    dst = (dst_iota + offsets + corr).reshape(-1)
    return src, dst


# speedup vs baseline: 1.0716x; 1.0716x over previous
import jax
import jax.numpy as jnp
from jax.experimental import pallas as pl
from jax.experimental.pallas import tpu as pltpu

_K = 9
_MAX_DILATION = 3
_KD = 18
_OUT_COLS = 16
_IPS = 1


def _knn_body(x_ref, out_ref, xx0, xx1, h0, h1):
    i = pl.program_id(0)
    n = x_ref.shape[1]
    rows = _IPS * n

    def produce(xx_s, h_s):
        for g in range(_IPS):
            xg = x_ref[g]
            xx_s[g * n:(g + 1) * n, :] = jax.lax.dot_general(
                xg, xg, (((1,), (1,)), ((), ())),
                preferred_element_type=jnp.float32,
            )
            h_s[g * n:(g + 1) * n, :] = 0.5 * jnp.sum(xg * xg, axis=1, keepdims=True)

    def consume(xx_s, h_s):
        h = h_s[...]
        row_mod = jax.lax.broadcasted_iota(jnp.int32, (rows, 1), 0) % n
        eye = row_mod == jax.lax.broadcasted_iota(jnp.int32, (rows, n), 1)
        ht = jnp.concatenate(
            [jnp.broadcast_to(jnp.transpose(h[g * n:(g + 1) * n]), (n, n))
             for g in range(_IPS)], axis=0)
        d2h = jnp.where(eye, 0.0, jnp.maximum(h + ht - xx_s[...], 0.0))
        iota_f = jax.lax.broadcasted_iota(jnp.int32, (rows, n), 1).astype(jnp.float32)
        big = jnp.float32(1e30)
        cols = [row_mod.astype(jnp.float32)]
        m = jnp.float32(0.0)
        for k in range(1, _KD):
            m = jnp.min(jnp.where(d2h > m, d2h, big), axis=1, keepdims=True)
            if k % 2 == 0:
                cols.append(
                    jnp.min(jnp.where(d2h == m, iota_f, big), axis=1, keepdims=True)
                )
        cols.append(jnp.zeros((rows, _OUT_COLS - len(cols)), jnp.float32))
        res = jnp.concatenate(cols, axis=1).astype(jnp.int32)
        out_ref[...] = res.reshape(_IPS, n, _OUT_COLS)

    @pl.when(i % 2 == 0)
    def _():
        produce(xx0, h0)
        consume(xx1, h1)

    @pl.when(i % 2 == 1)
    def _():
        produce(xx1, h1)
        consume(xx0, h0)


def kernel(x, layer_idx):
    B, N, C = x.shape
    nsteps = B // _IPS
    idx_pad = pl.pallas_call(
        _knn_body,
        grid=(nsteps + 1,),
        in_specs=[pl.BlockSpec(
            (_IPS, N, C), lambda i: (jnp.minimum(i, nsteps - 1), 0, 0))],
        out_specs=pl.BlockSpec(
            (_IPS, N, _OUT_COLS), lambda i: (jnp.maximum(i - 1, 0), 0, 0)),
        out_shape=jax.ShapeDtypeStruct((B, N, _OUT_COLS), jnp.int32),
        scratch_shapes=[
            pltpu.VMEM((_IPS * N, N), jnp.float32),
            pltpu.VMEM((_IPS * N, N), jnp.float32),
            pltpu.VMEM((_IPS * N, 1), jnp.float32),
            pltpu.VMEM((_IPS * N, 1), jnp.float32),
        ],
    )(x)
    idx9 = idx_pad[:, :, :_K]
    dil_traced = jnp.minimum(layer_idx // 4 + 1, _MAX_DILATION)
    corr = (dil_traced - 2).astype(jnp.int32)
    offsets = (jnp.arange(B, dtype=jnp.int32) * N)[:, None, None]
    src = (idx9 + offsets + corr).reshape(-1)
    dst_iota = jnp.broadcast_to(
        jnp.arange(N, dtype=jnp.int32)[None, :, None], (B, N, _K)
    )
    dst = (dst_iota + offsets + corr).reshape(-1)
    return src, dst
